# baseline (speedup 1.0000x reference)
import functools

import jax
import jax.numpy as jnp
from jax import lax
from jax.experimental import pallas as pl
from jax.experimental.pallas import tpu as pltpu
from jax.experimental.pallas import tpu_sc as plsc

V = 1_000_000
D = 32
B = 16384 * 26
NC = 2
NS = 16
NW = NC * NS
CHUNK = 128
GRP = 8
GROUP = GRP * CHUNK
B_PER_W = B // NW
NG = B_PER_W // GROUP

assert B_PER_W * NW == B and NG * GROUP == B_PER_W


@functools.partial(
    pl.kernel,
    mesh=plsc.VectorSubcoreMesh(core_axis_name="c", subcore_axis_name="s"),
    out_type=jax.ShapeDtypeStruct((B, D), jnp.float32),
    scratch_types=[
        pltpu.VMEM((GROUP,), jnp.int32),
        pltpu.VMEM((GROUP, D), jnp.float32),
        pltpu.SemaphoreType.DMA,
    ],
    compiler_params=pltpu.CompilerParams(use_tc_tiling_on_sc=False),
)
def _gather_kernel(idx_hbm, table_hbm, out_hbm, idx_v, rows_v, sem):
    wid = lax.axis_index("s") * NC + lax.axis_index("c")
    base = wid * B_PER_W

    def group_body(g, carry):
        pltpu.sync_copy(idx_hbm.at[pl.ds(base + g * GROUP, GROUP)], idx_v)
        pltpu.async_copy(table_hbm.at[idx_v], rows_v, sem).wait()
        pltpu.sync_copy(rows_v, out_hbm.at[pl.ds(base + g * GROUP, GROUP)])
        return carry

    lax.fori_loop(0, NG, group_body, 0)


def kernel(input, others):
    idx = input.astype(jnp.int32).reshape(B)
    out = _gather_kernel(idx, others)
    return out.reshape(input.shape[0], input.shape[1], D)


# speedup vs baseline: 1.0154x; 1.0154x over previous
import functools

import jax
import jax.numpy as jnp
from jax import lax
from jax.experimental import pallas as pl
from jax.experimental.pallas import tpu as pltpu
from jax.experimental.pallas import tpu_sc as plsc

V = 1_000_000
D = 32
NB = 16384
NF = 26
NC = 2
NS = 16
NW = NC * NS
ROWS_W = NB // NW
GR = 64
NG = ROWS_W // GR

assert NW * ROWS_W == NB and NG * GR == ROWS_W


@functools.partial(
    pl.kernel,
    mesh=plsc.VectorSubcoreMesh(core_axis_name="c", subcore_axis_name="s"),
    out_type=jax.ShapeDtypeStruct((NB, NF, D), jnp.float32),
    scratch_types=[
        pltpu.VMEM((GR, NF), jnp.int32),
        pltpu.VMEM((GR, NF), jnp.int32),
        pltpu.VMEM((GR, NF, D), jnp.float32),
        pltpu.VMEM((GR, NF, D), jnp.float32),
        pltpu.SemaphoreType.DMA,
        pltpu.SemaphoreType.DMA,
        pltpu.SemaphoreType.DMA,
        pltpu.SemaphoreType.DMA,
    ],
    compiler_params=pltpu.CompilerParams(use_tc_tiling_on_sc=False),
)
def _gather_kernel(idx_hbm, table_hbm, out_hbm,
                   idx0, idx1, rows0, rows1, g0, g1, o0, o1):
    wid = lax.axis_index("s") * NC + lax.axis_index("c")
    b0 = wid * ROWS_W

    idxs = (idx0, idx1)
    rows = (rows0, rows1)
    gsem = (g0, g1)
    osem = (o0, o1)

    def stage_and_fire(g, s):
        pltpu.sync_copy(idx_hbm.at[pl.ds(b0 + g * GR, GR)], idxs[s])

        def fire(i, c):
            pltpu.async_copy(
                table_hbm.at[idxs[s].at[i]],
                rows[s].at[i],
                gsem[s],
            )
            return c

        lax.fori_loop(0, GR, fire, 0)

    def wait_gathers(s):
        def w(i, c):
            pltpu.make_async_copy(
                table_hbm.at[idxs[s].at[0]],
                rows[s].at[0],
                gsem[s],
            ).wait()
            return c

        lax.fori_loop(0, GR, w, 0)

    def fire_out(g, s):
        pltpu.async_copy(rows[s], out_hbm.at[pl.ds(b0 + g * GR, GR)], osem[s])

    def wait_out(s):
        pltpu.make_async_copy(
            rows[s], out_hbm.at[pl.ds(b0, GR)], osem[s]
        ).wait()

    stage_and_fire(0, 0)

    def body(i, carry):
        for s in (0, 1):
            k = 2 * i + s
            nxt = k + 1

            @pl.when(nxt < NG)
            def _():
                @pl.when(nxt >= 2)
                def _():
                    wait_out(1 - s)
                stage_and_fire(nxt, 1 - s)

            wait_gathers(s)
            fire_out(k, s)
        return carry

    lax.fori_loop(0, NG // 2, body, 0)
    wait_out(0)
    wait_out(1)


def kernel(input, others):
    return _gather_kernel(input.astype(jnp.int32), others)
